# baseline (speedup 1.0000x reference)
import functools

import jax
import jax.numpy as jnp
from jax import lax
from jax.experimental import pallas as pl
from jax.experimental.pallas import tpu as pltpu
from jax.experimental.pallas import tpu_sc as plsc

_NC = 2
_NS = 16
_NW = _NC * _NS
_CH = 80


_HIMASK = -65536
_HALF = 32768


def _pack_bf16_pair(v):
    k = v.shape[-1] // 2
    ilo = lax.bitcast_convert_type(v[:, :k], jnp.int32)
    ihi = lax.bitcast_convert_type(v[:, k:], jnp.int32)
    lo16 = lax.shift_right_logical(ilo + _HALF, 16)
    hi16 = (ihi + _HALF) & _HIMASK
    return hi16 | lo16


def _pq_body(x_ref, wa_ref, wb_ref, p_ref, q_ref):
    xb = x_ref[...]
    p_ref[...] = _pack_bf16_pair(
        jnp.dot(xb, wa_ref[...], preferred_element_type=jnp.float32))
    q_ref[...] = _pack_bf16_pair(
        jnp.dot(xb, wb_ref[...], preferred_element_type=jnp.float32))


def _edge_body(e_ref, g_ref, w1_ref, b1_ref, w2_ref, b2_ref, out_ref):
    g32 = g_ref[...]
    glo = lax.bitcast_convert_type(jnp.left_shift(g32, 16), jnp.float32)
    ghi = lax.bitcast_convert_type(g32 & _HIMASK, jnp.float32)
    gperm = jnp.concatenate([glo, ghi], axis=-1)
    h = jnp.dot(e_ref[...], w1_ref[...], preferred_element_type=jnp.float32)
    h = jnp.maximum(h + gperm + b1_ref[...], 0.0)
    out_ref[...] = jnp.dot(h, w2_ref[...], preferred_element_type=jnp.float32) + b2_ref[...]


def _node_body(x_ref, *refs):
    w1a_ref, w1b_ref, b1_ref, w2_ref, b2_ref, out_ref = refs[-6:]
    agg_refs = refs[:-6]
    agg = agg_refs[0][...]
    for a_ref in agg_refs[1:]:
        agg = agg + a_ref[...]
    h = jnp.dot(x_ref[...], w1a_ref[...], preferred_element_type=jnp.float32)
    h = h + jnp.dot(agg, w1b_ref[...], preferred_element_type=jnp.float32)
    h = jnp.maximum(h + b1_ref[...], 0.0)
    out_ref[...] = jnp.dot(h, w2_ref[...], preferred_element_type=jnp.float32) + b2_ref[...]


def _gather_body(p_hbm, q_hbm, src_hbm, dst_hbm, g_hbm,
                 idx_s0, idx_d0, buf_a0, buf_b0,
                 idx_s1, idx_d1, buf_a1, buf_b1,
                 s_is0, s_id0, s_a0, s_b0, s_w0,
                 s_is1, s_id1, s_a1, s_b1, s_w1, *, epw, nch, h):
    c = lax.axis_index("c")
    s = lax.axis_index("s")
    base = (s * _NC + c) * epw
    slots = (
        dict(idx_s=idx_s0, idx_d=idx_d0, buf_a=buf_a0, buf_b=buf_b0,
             s_is=s_is0, s_id=s_id0, s_a=s_a0, s_b=s_b0, s_w=s_w0),
        dict(idx_s=idx_s1, idx_d=idx_d1, buf_a=buf_a1, buf_b=buf_b1,
             s_is=s_is1, s_id=s_id1, s_a=s_a1, s_b=s_b1, s_w=s_w1),
    )

    def start_l(k, sl):
        off = base + k * _CH
        pltpu.async_copy(src_hbm.at[pl.ds(off, _CH)], sl["idx_s"], sl["s_is"])
        pltpu.async_copy(dst_hbm.at[pl.ds(off, _CH)], sl["idx_d"], sl["s_id"])

    def wait_l(sl):
        pltpu.make_async_copy(src_hbm.at[pl.ds(base, _CH)], sl["idx_s"],
                              sl["s_is"]).wait()
        pltpu.make_async_copy(dst_hbm.at[pl.ds(base, _CH)], sl["idx_d"],
                              sl["s_id"]).wait()

    def start_g(sl):
        pltpu.async_copy(p_hbm.at[sl["idx_s"]], sl["buf_a"], sl["s_a"])
        pltpu.async_copy(q_hbm.at[sl["idx_d"]], sl["buf_b"], sl["s_b"])

    def wait_g(sl):
        pltpu.make_async_copy(p_hbm.at[sl["idx_s"]], sl["buf_a"],
                              sl["s_a"]).wait()
        pltpu.make_async_copy(q_hbm.at[sl["idx_d"]], sl["buf_b"],
                              sl["s_b"]).wait()

    def start_w(k, sl):
        pltpu.async_copy(sl["buf_a"], g_hbm.at[pl.ds(base + k * _CH, _CH)],
                         sl["s_w"])

    def wait_w(sl):
        pltpu.make_async_copy(sl["buf_a"], g_hbm.at[pl.ds(base, _CH)],
                              sl["s_w"]).wait()

    def vadd(sl):
        buf_a, buf_b = sl["buf_a"], sl["buf_b"]
        hw = h // 2

        def row(r, rc):
            for cc in range(0, hw, 16):
                va = buf_a[r, pl.ds(cc, 16)]
                vb = buf_b[r, pl.ds(cc, 16)]
                slo = (lax.bitcast_convert_type(jnp.left_shift(va, 16),
                                                jnp.float32)
                       + lax.bitcast_convert_type(jnp.left_shift(vb, 16),
                                                  jnp.float32))
                shi = (lax.bitcast_convert_type(va & _HIMASK, jnp.float32)
                       + lax.bitcast_convert_type(vb & _HIMASK, jnp.float32))
                ilo = lax.bitcast_convert_type(slo, jnp.int32)
                ihi = lax.bitcast_convert_type(shi, jnp.int32)
                buf_a[r, pl.ds(cc, 16)] = (
                    ((ihi + _HALF) & _HIMASK)
                    | lax.shift_right_logical(ilo + _HALF, 16))
            return rc

        lax.fori_loop(0, _CH, row, 0)

    start_l(0, slots[0])
    wait_l(slots[0])
    start_g(slots[0])
    start_l(1, slots[1])

    def step(k, sl, osl):
        @pl.when(k + 1 < nch)
        def _():
            wait_l(osl)

        @pl.when(k >= 1)
        def _():
            wait_w(osl)

        @pl.when(k + 1 < nch)
        def _():
            start_g(osl)

        wait_g(sl)

        @pl.when(k + 2 < nch)
        def _():
            start_l(k + 2, sl)

        vadd(sl)
        start_w(k, sl)

    def pair(i2, carry):
        k0 = 2 * i2

        @pl.when(k0 < nch)
        def _():
            step(k0, slots[0], slots[1])

        @pl.when(k0 + 1 < nch)
        def _():
            step(k0 + 1, slots[1], slots[0])

        return carry

    lax.fori_loop(0, (nch + 1) // 2, pair, 0)
    wait_w(slots[(nch - 1) % 2])


def _scatter_body(en_hbm, dst_hbm, aggp_hbm, agg_sh,
                  idx0, rows0, idx1, rows1,
                  s_i0, s_r0, s_s0, s_i1, s_r1, s_s1,
                  *, epw, nch, n, h):
    c = lax.axis_index("c")
    s = lax.axis_index("s")
    wid = s * _NC + c
    base = wid * epw
    slots = (
        dict(idx=idx0, rows=rows0, s_i=s_i0, s_r=s_r0, s_s=s_s0),
        dict(idx=idx1, rows=rows1, s_i=s_i1, s_r=s_r1, s_s=s_s1),
    )
    nzc = n // _CH
    zper = (nzc + _NS - 1) // _NS

    def zrow(r, rc):
        for cc in range(0, h, 16):
            rows0[r, pl.ds(cc, 16)] = jnp.zeros((16,), jnp.float32)
        return rc

    lax.fori_loop(0, _CH, zrow, 0)

    def zcp(j, rc):
        k = s + j * _NS

        @pl.when(k < nzc)
        def _():
            pltpu.sync_copy(rows0, agg_sh.at[pl.ds(k * _CH, _CH)])

        return rc

    lax.fori_loop(0, zper, zcp, 0)
    plsc.subcore_barrier()

    def start_l(k, sl):
        off = base + k * _CH
        pltpu.async_copy(dst_hbm.at[pl.ds(off, _CH)], sl["idx"], sl["s_i"])
        pltpu.async_copy(en_hbm.at[pl.ds(off, _CH)], sl["rows"], sl["s_r"])

    def wait_l(sl):
        pltpu.make_async_copy(dst_hbm.at[pl.ds(base, _CH)], sl["idx"],
                              sl["s_i"]).wait()
        pltpu.make_async_copy(en_hbm.at[pl.ds(base, _CH)], sl["rows"],
                              sl["s_r"]).wait()

    def start_s(sl):
        pltpu.async_copy(sl["rows"], agg_sh.at[sl["idx"]], sl["s_s"],
                         add=True)

    def wait_s(sl):
        pltpu.make_async_copy(sl["rows"], agg_sh.at[sl["idx"]],
                              sl["s_s"]).wait()

    start_l(0, slots[0])

    def step(k, sl, osl):
        wait_l(sl)
        start_s(sl)

        @pl.when(k >= 1)
        def _():
            wait_s(osl)

        @pl.when(k + 1 < nch)
        def _():
            start_l(k + 1, osl)

    def pair(i2, carry):
        k0 = 2 * i2

        @pl.when(k0 < nch)
        def _():
            step(k0, slots[0], slots[1])

        @pl.when(k0 + 1 < nch)
        def _():
            step(k0 + 1, slots[1], slots[0])

        return carry

    lax.fori_loop(0, (nch + 1) // 2, pair, 0)
    wait_s(slots[(nch - 1) % 2])
    plsc.subcore_barrier()

    def wcp(j, rc):
        k = s + j * _NS

        @pl.when(k < nzc)
        def _():
            pltpu.sync_copy(agg_sh.at[pl.ds(k * _CH, _CH)],
                            aggp_hbm.at[pl.ds(c * n + k * _CH, _CH)])

        return rc

    lax.fori_loop(0, zper, wcp, 0)


def kernel(x, edge_index, e, Wb1, bb1, Wb2, bb2, Wa1, ba1, Wa2, ba2):
    n, h = x.shape
    eN = e.shape[0]
    assert eN % _NW == 0
    epw = eN // _NW
    assert epw % _CH == 0
    nch = epw // _CH
    assert n % _NS == 0 and h % 16 == 0

    src = edge_index[0]
    dst = edge_index[1]
    hw = h // 2
    perm = list(range(0, h, 2)) + list(range(1, h, 2))
    Wb1a, Wb1b, Wb1c = Wb1[:h, perm], Wb1[h:2 * h, perm], Wb1[2 * h:, perm]
    Wb2p = Wb2[perm, :]
    Wa1a, Wa1b = Wa1[:h], Wa1[h:]
    bb1r = bb1[jnp.array(perm)].reshape(1, h)
    bb2r = bb2.reshape(1, h)
    ba1r = ba1.reshape(1, h)
    ba2r = ba2.reshape(1, h)

    nb = 1000
    Ppk, Qpk = pl.pallas_call(
        _pq_body,
        grid=(n // nb,),
        in_specs=[
            pl.BlockSpec((nb, h), lambda i: (i, 0)),
            pl.BlockSpec((h, h), lambda i: (0, 0)),
            pl.BlockSpec((h, h), lambda i: (0, 0)),
        ],
        out_specs=[
            pl.BlockSpec((nb, hw), lambda i: (i, 0)),
            pl.BlockSpec((nb, hw), lambda i: (i, 0)),
        ],
        out_shape=[
            jax.ShapeDtypeStruct((n, hw), jnp.int32),
            jax.ShapeDtypeStruct((n, hw), jnp.int32),
        ],
    )(x, Wb1a, Wb1b)

    mesh = plsc.VectorSubcoreMesh(core_axis_name="c", subcore_axis_name="s")
    gather_k = pl.kernel(
        functools.partial(_gather_body, epw=epw, nch=nch, h=h),
        out_type=jax.ShapeDtypeStruct((eN, hw), jnp.int32),
        mesh=mesh,
        compiler_params=pltpu.CompilerParams(use_tc_tiling_on_sc=False),
        scratch_types=(
            [pltpu.VMEM((_CH,), jnp.int32), pltpu.VMEM((_CH,), jnp.int32),
             pltpu.VMEM((_CH, hw), jnp.int32), pltpu.VMEM((_CH, hw), jnp.int32)] * 2
            + [pltpu.SemaphoreType.DMA] * 10
        ),
    )
    Gpk = gather_k(Ppk, Qpk, src, dst)

    eb = 3200
    e_new = pl.pallas_call(
        _edge_body,
        grid=(eN // eb,),
        in_specs=[
            pl.BlockSpec((eb, h), lambda i: (i, 0)),
            pl.BlockSpec((eb, hw), lambda i: (i, 0)),
            pl.BlockSpec((h, h), lambda i: (0, 0)),
            pl.BlockSpec((1, h), lambda i: (0, 0)),
            pl.BlockSpec((h, h), lambda i: (0, 0)),
            pl.BlockSpec((1, h), lambda i: (0, 0)),
        ],
        out_specs=pl.BlockSpec((eb, h), lambda i: (i, 0)),
        out_shape=jax.ShapeDtypeStruct((eN, h), jnp.float32),
    )(e, Gpk, Wb1c, bb1r, Wb2p, bb2r)

    scatter_k = pl.kernel(
        functools.partial(_scatter_body, epw=epw, nch=nch, n=n, h=h),
        out_type=jax.ShapeDtypeStruct((_NC * n, h), jnp.float32),
        mesh=mesh,
        scratch_types=(
            [pltpu.VMEM_SHARED((n, h), jnp.float32)]
            + [pltpu.VMEM((_CH,), jnp.int32), pltpu.VMEM((_CH, h), jnp.float32)] * 2
            + [pltpu.SemaphoreType.DMA] * 6
        ),
    )
    aggp = scatter_k(e_new, dst)
    agg_parts = [aggp[:n], aggp[n:]]

    x_new = pl.pallas_call(
        _node_body,
        grid=(n // nb,),
        in_specs=[pl.BlockSpec((nb, h), lambda i: (i, 0))]
        + [pl.BlockSpec((nb, h), lambda i: (i, 0))] * len(agg_parts)
        + [
            pl.BlockSpec((h, h), lambda i: (0, 0)),
            pl.BlockSpec((h, h), lambda i: (0, 0)),
            pl.BlockSpec((1, h), lambda i: (0, 0)),
            pl.BlockSpec((h, h), lambda i: (0, 0)),
            pl.BlockSpec((1, h), lambda i: (0, 0)),
        ],
        out_specs=pl.BlockSpec((nb, h), lambda i: (i, 0)),
        out_shape=jax.ShapeDtypeStruct((n, h), jnp.float32),
    )(x, *agg_parts, Wa1a, Wa1b, ba1r, Wa2, ba2r)

    return (x_new, e_new)


# speedup vs baseline: 1.0110x; 1.0110x over previous
import functools

import jax
import jax.numpy as jnp
from jax import lax
from jax.experimental import pallas as pl
from jax.experimental.pallas import tpu as pltpu
from jax.experimental.pallas import tpu_sc as plsc

_NC = 2
_NS = 16
_NW = _NC * _NS
_CH = 80


_HIMASK = -65536
_HALF = 32768


def _pack_bf16_pair(v):
    k = v.shape[-1] // 2
    ilo = lax.bitcast_convert_type(v[:, :k], jnp.int32)
    ihi = lax.bitcast_convert_type(v[:, k:], jnp.int32)
    lo16 = lax.shift_right_logical(ilo + _HALF, 16)
    hi16 = (ihi + _HALF) & _HIMASK
    return hi16 | lo16


def _pq_body(x_ref, wa_ref, wb_ref, p_ref, q_ref):
    xb = x_ref[...]
    p_ref[...] = _pack_bf16_pair(
        jnp.dot(xb, wa_ref[...], preferred_element_type=jnp.float32))
    q_ref[...] = _pack_bf16_pair(
        jnp.dot(xb, wb_ref[...], preferred_element_type=jnp.float32))


def _edge_body(e_ref, g_ref, w1_ref, b1_ref, w2_ref, b2_ref, out_ref):
    h = jnp.dot(e_ref[...], w1_ref[...], preferred_element_type=jnp.float32)
    h = jnp.maximum(h + g_ref[...] + b1_ref[...], 0.0)
    out_ref[...] = jnp.dot(h, w2_ref[...], preferred_element_type=jnp.float32) + b2_ref[...]


def _node_body(x_ref, *refs):
    w1a_ref, w1b_ref, b1_ref, w2_ref, b2_ref, out_ref = refs[-6:]
    agg_refs = refs[:-6]
    agg = agg_refs[0][...]
    for a_ref in agg_refs[1:]:
        agg = agg + a_ref[...]
    h = jnp.dot(x_ref[...], w1a_ref[...], preferred_element_type=jnp.float32)
    h = h + jnp.dot(agg, w1b_ref[...], preferred_element_type=jnp.float32)
    h = jnp.maximum(h + b1_ref[...], 0.0)
    out_ref[...] = jnp.dot(h, w2_ref[...], preferred_element_type=jnp.float32) + b2_ref[...]


def _gather_body(p_hbm, q_hbm, src_hbm, dst_hbm, g_hbm,
                 idx_s0, idx_d0, buf_a0, buf_b0, buf_o0,
                 idx_s1, idx_d1, buf_a1, buf_b1, buf_o1,
                 s_is0, s_id0, s_a0, s_b0, s_w0,
                 s_is1, s_id1, s_a1, s_b1, s_w1, *, epw, nch, h):
    c = lax.axis_index("c")
    s = lax.axis_index("s")
    base = (s * _NC + c) * epw
    slots = (
        dict(idx_s=idx_s0, idx_d=idx_d0, buf_a=buf_a0, buf_b=buf_b0,
             buf_o=buf_o0,
             s_is=s_is0, s_id=s_id0, s_a=s_a0, s_b=s_b0, s_w=s_w0),
        dict(idx_s=idx_s1, idx_d=idx_d1, buf_a=buf_a1, buf_b=buf_b1,
             buf_o=buf_o1,
             s_is=s_is1, s_id=s_id1, s_a=s_a1, s_b=s_b1, s_w=s_w1),
    )

    def start_l(k, sl):
        off = base + k * _CH
        pltpu.async_copy(src_hbm.at[pl.ds(off, _CH)], sl["idx_s"], sl["s_is"])
        pltpu.async_copy(dst_hbm.at[pl.ds(off, _CH)], sl["idx_d"], sl["s_id"])

    def wait_l(sl):
        pltpu.make_async_copy(src_hbm.at[pl.ds(base, _CH)], sl["idx_s"],
                              sl["s_is"]).wait()
        pltpu.make_async_copy(dst_hbm.at[pl.ds(base, _CH)], sl["idx_d"],
                              sl["s_id"]).wait()

    def start_g(sl):
        pltpu.async_copy(p_hbm.at[sl["idx_s"]], sl["buf_a"], sl["s_a"])
        pltpu.async_copy(q_hbm.at[sl["idx_d"]], sl["buf_b"], sl["s_b"])

    def wait_g(sl):
        pltpu.make_async_copy(p_hbm.at[sl["idx_s"]], sl["buf_a"],
                              sl["s_a"]).wait()
        pltpu.make_async_copy(q_hbm.at[sl["idx_d"]], sl["buf_b"],
                              sl["s_b"]).wait()

    def start_w(k, sl):
        pltpu.async_copy(sl["buf_o"], g_hbm.at[pl.ds(base + k * _CH, _CH)],
                         sl["s_w"])

    def wait_w(sl):
        pltpu.make_async_copy(sl["buf_o"], g_hbm.at[pl.ds(base, _CH)],
                              sl["s_w"]).wait()

    def vadd(sl):
        buf_a, buf_b, buf_o = sl["buf_a"], sl["buf_b"], sl["buf_o"]
        hw = h // 2

        def row(r, rc):
            for cc in range(0, hw, 16):
                va = buf_a[r, pl.ds(cc, 16)]
                vb = buf_b[r, pl.ds(cc, 16)]
                buf_o[r, pl.ds(cc, 16)] = (
                    lax.bitcast_convert_type(jnp.left_shift(va, 16),
                                             jnp.float32)
                    + lax.bitcast_convert_type(jnp.left_shift(vb, 16),
                                               jnp.float32))
                buf_o[r, pl.ds(hw + cc, 16)] = (
                    lax.bitcast_convert_type(va & _HIMASK, jnp.float32)
                    + lax.bitcast_convert_type(vb & _HIMASK, jnp.float32))
            return rc

        lax.fori_loop(0, _CH, row, 0)

    start_l(0, slots[0])
    wait_l(slots[0])
    start_g(slots[0])
    start_l(1, slots[1])

    def step(k, sl, osl):
        @pl.when(k + 1 < nch)
        def _():
            wait_l(osl)

        @pl.when(k + 1 < nch)
        def _():
            start_g(osl)

        wait_g(sl)

        @pl.when(k + 2 < nch)
        def _():
            start_l(k + 2, sl)

        @pl.when(k >= 2)
        def _():
            wait_w(sl)

        vadd(sl)
        start_w(k, sl)

    def pair(i2, carry):
        k0 = 2 * i2

        @pl.when(k0 < nch)
        def _():
            step(k0, slots[0], slots[1])

        @pl.when(k0 + 1 < nch)
        def _():
            step(k0 + 1, slots[1], slots[0])

        return carry

    lax.fori_loop(0, (nch + 1) // 2, pair, 0)
    if nch >= 2:
        wait_w(slots[nch % 2])
    wait_w(slots[(nch - 1) % 2])


def _scatter_body(en_hbm, dst_hbm, aggp_hbm, agg_sh,
                  idx0, rows0, idx1, rows1,
                  s_i0, s_r0, s_s0, s_i1, s_r1, s_s1,
                  *, epw, nch, n, h):
    c = lax.axis_index("c")
    s = lax.axis_index("s")
    wid = s * _NC + c
    base = wid * epw
    slots = (
        dict(idx=idx0, rows=rows0, s_i=s_i0, s_r=s_r0, s_s=s_s0),
        dict(idx=idx1, rows=rows1, s_i=s_i1, s_r=s_r1, s_s=s_s1),
    )
    nzc = n // _CH
    zper = (nzc + _NS - 1) // _NS

    def zrow(r, rc):
        for cc in range(0, h, 16):
            rows0[r, pl.ds(cc, 16)] = jnp.zeros((16,), jnp.float32)
        return rc

    lax.fori_loop(0, _CH, zrow, 0)

    def zcp(j, rc):
        k = s + j * _NS

        @pl.when(k < nzc)
        def _():
            pltpu.sync_copy(rows0, agg_sh.at[pl.ds(k * _CH, _CH)])

        return rc

    lax.fori_loop(0, zper, zcp, 0)
    plsc.subcore_barrier()

    def start_l(k, sl):
        off = base + k * _CH
        pltpu.async_copy(dst_hbm.at[pl.ds(off, _CH)], sl["idx"], sl["s_i"])
        pltpu.async_copy(en_hbm.at[pl.ds(off, _CH)], sl["rows"], sl["s_r"])

    def wait_l(sl):
        pltpu.make_async_copy(dst_hbm.at[pl.ds(base, _CH)], sl["idx"],
                              sl["s_i"]).wait()
        pltpu.make_async_copy(en_hbm.at[pl.ds(base, _CH)], sl["rows"],
                              sl["s_r"]).wait()

    def start_s(sl):
        pltpu.async_copy(sl["rows"], agg_sh.at[sl["idx"]], sl["s_s"],
                         add=True)

    def wait_s(sl):
        pltpu.make_async_copy(sl["rows"], agg_sh.at[sl["idx"]],
                              sl["s_s"]).wait()

    start_l(0, slots[0])

    def step(k, sl, osl):
        wait_l(sl)
        start_s(sl)

        @pl.when(k >= 1)
        def _():
            wait_s(osl)

        @pl.when(k + 1 < nch)
        def _():
            start_l(k + 1, osl)

    def pair(i2, carry):
        k0 = 2 * i2

        @pl.when(k0 < nch)
        def _():
            step(k0, slots[0], slots[1])

        @pl.when(k0 + 1 < nch)
        def _():
            step(k0 + 1, slots[1], slots[0])

        return carry

    lax.fori_loop(0, (nch + 1) // 2, pair, 0)
    wait_s(slots[(nch - 1) % 2])
    plsc.subcore_barrier()

    def wcp(j, rc):
        k = s + j * _NS

        @pl.when(k < nzc)
        def _():
            pltpu.sync_copy(agg_sh.at[pl.ds(k * _CH, _CH)],
                            aggp_hbm.at[pl.ds(c * n + k * _CH, _CH)])

        return rc

    lax.fori_loop(0, zper, wcp, 0)


def kernel(x, edge_index, e, Wb1, bb1, Wb2, bb2, Wa1, ba1, Wa2, ba2):
    n, h = x.shape
    eN = e.shape[0]
    assert eN % _NW == 0
    epw = eN // _NW
    assert epw % _CH == 0
    nch = epw // _CH
    assert n % _NS == 0 and h % 16 == 0

    src = edge_index[0]
    dst = edge_index[1]
    hw = h // 2
    perm = list(range(0, h, 2)) + list(range(1, h, 2))
    Wb1a, Wb1b, Wb1c = Wb1[:h, perm], Wb1[h:2 * h, perm], Wb1[2 * h:, perm]
    Wb2p = Wb2[perm, :]
    Wa1a, Wa1b = Wa1[:h], Wa1[h:]
    bb1r = bb1[jnp.array(perm)].reshape(1, h)
    bb2r = bb2.reshape(1, h)
    ba1r = ba1.reshape(1, h)
    ba2r = ba2.reshape(1, h)

    nb = 1000
    Ppk, Qpk = pl.pallas_call(
        _pq_body,
        grid=(n // nb,),
        in_specs=[
            pl.BlockSpec((nb, h), lambda i: (i, 0)),
            pl.BlockSpec((h, h), lambda i: (0, 0)),
            pl.BlockSpec((h, h), lambda i: (0, 0)),
        ],
        out_specs=[
            pl.BlockSpec((nb, hw), lambda i: (i, 0)),
            pl.BlockSpec((nb, hw), lambda i: (i, 0)),
        ],
        out_shape=[
            jax.ShapeDtypeStruct((n, hw), jnp.int32),
            jax.ShapeDtypeStruct((n, hw), jnp.int32),
        ],
    )(x, Wb1a, Wb1b)

    mesh = plsc.VectorSubcoreMesh(core_axis_name="c", subcore_axis_name="s")
    gather_k = pl.kernel(
        functools.partial(_gather_body, epw=epw, nch=nch, h=h),
        out_type=jax.ShapeDtypeStruct((eN, h), jnp.float32),
        mesh=mesh,
        compiler_params=pltpu.CompilerParams(use_tc_tiling_on_sc=False),
        scratch_types=(
            [pltpu.VMEM((_CH,), jnp.int32), pltpu.VMEM((_CH,), jnp.int32),
             pltpu.VMEM((_CH, hw), jnp.int32), pltpu.VMEM((_CH, hw), jnp.int32),
             pltpu.VMEM((_CH, h), jnp.float32)] * 2
            + [pltpu.SemaphoreType.DMA] * 10
        ),
    )
    G = gather_k(Ppk, Qpk, src, dst)

    eb = 3200
    e_new = pl.pallas_call(
        _edge_body,
        grid=(eN // eb,),
        in_specs=[
            pl.BlockSpec((eb, h), lambda i: (i, 0)),
            pl.BlockSpec((eb, h), lambda i: (i, 0)),
            pl.BlockSpec((h, h), lambda i: (0, 0)),
            pl.BlockSpec((1, h), lambda i: (0, 0)),
            pl.BlockSpec((h, h), lambda i: (0, 0)),
            pl.BlockSpec((1, h), lambda i: (0, 0)),
        ],
        out_specs=pl.BlockSpec((eb, h), lambda i: (i, 0)),
        out_shape=jax.ShapeDtypeStruct((eN, h), jnp.float32),
    )(e, G, Wb1c, bb1r, Wb2p, bb2r)

    scatter_k = pl.kernel(
        functools.partial(_scatter_body, epw=epw, nch=nch, n=n, h=h),
        out_type=jax.ShapeDtypeStruct((_NC * n, h), jnp.float32),
        mesh=mesh,
        scratch_types=(
            [pltpu.VMEM_SHARED((n, h), jnp.float32)]
            + [pltpu.VMEM((_CH,), jnp.int32), pltpu.VMEM((_CH, h), jnp.float32)] * 2
            + [pltpu.SemaphoreType.DMA] * 6
        ),
    )
    aggp = scatter_k(e_new, dst)
    agg_parts = [aggp[:n], aggp[n:]]

    x_new = pl.pallas_call(
        _node_body,
        grid=(n // nb,),
        in_specs=[pl.BlockSpec((nb, h), lambda i: (i, 0))]
        + [pl.BlockSpec((nb, h), lambda i: (i, 0))] * len(agg_parts)
        + [
            pl.BlockSpec((h, h), lambda i: (0, 0)),
            pl.BlockSpec((h, h), lambda i: (0, 0)),
            pl.BlockSpec((1, h), lambda i: (0, 0)),
            pl.BlockSpec((h, h), lambda i: (0, 0)),
            pl.BlockSpec((1, h), lambda i: (0, 0)),
        ],
        out_specs=pl.BlockSpec((nb, h), lambda i: (i, 0)),
        out_shape=jax.ShapeDtypeStruct((n, h), jnp.float32),
    )(x, *agg_parts, Wa1a, Wa1b, ba1r, Wa2, ba2r)

    return (x_new, e_new)


# speedup vs baseline: 2.8573x; 2.8261x over previous
import functools

import jax
import jax.numpy as jnp
from jax import lax
from jax.experimental import pallas as pl
from jax.experimental.pallas import tpu as pltpu
from jax.experimental.pallas import tpu_sc as plsc

_NC = 2
_NS = 16
_NW = _NC * _NS
_CH = 80


def _pq_body(x_ref, wa_ref, wb_ref, p_ref, q_ref):
    xb = x_ref[...]
    p_ref[...] = jnp.dot(xb, wa_ref[...], preferred_element_type=jnp.float32)
    q_ref[...] = jnp.dot(xb, wb_ref[...], preferred_element_type=jnp.float32)


def _edge_body(e_ref, g_ref, w1_ref, b1_ref, w2_ref, b2_ref, out_ref):
    h = jnp.dot(e_ref[...], w1_ref[...], preferred_element_type=jnp.float32)
    h = jnp.maximum(h + g_ref[...] + b1_ref[...], 0.0)
    out_ref[...] = jnp.dot(h, w2_ref[...], preferred_element_type=jnp.float32) + b2_ref[...]


def _node_body(x_ref, *refs):
    w1a_ref, w1b_ref, b1_ref, w2_ref, b2_ref, out_ref = refs[-6:]
    agg_refs = refs[:-6]
    agg = agg_refs[0][...]
    for a_ref in agg_refs[1:]:
        agg = agg + a_ref[...]
    h = jnp.dot(x_ref[...], w1a_ref[...], preferred_element_type=jnp.float32)
    h = h + jnp.dot(agg, w1b_ref[...], preferred_element_type=jnp.float32)
    h = jnp.maximum(h + b1_ref[...], 0.0)
    out_ref[...] = jnp.dot(h, w2_ref[...], preferred_element_type=jnp.float32) + b2_ref[...]


def _gather_body(p_hbm, q_hbm, src_hbm, dst_hbm, g_hbm,
                 idx_s0, idx_d0, buf_a0, buf_b0, buf_o0,
                 idx_s1, idx_d1, buf_a1, buf_b1, buf_o1,
                 s_is0, s_id0, s_a0, s_b0, s_w0,
                 s_is1, s_id1, s_a1, s_b1, s_w1, *, epw, nch, h):
    c = lax.axis_index("c")
    s = lax.axis_index("s")
    base = (s * _NC + c) * epw
    slots = (
        dict(idx_s=idx_s0, idx_d=idx_d0, buf_a=buf_a0, buf_b=buf_b0,
             buf_o=buf_o0,
             s_is=s_is0, s_id=s_id0, s_a=s_a0, s_b=s_b0, s_w=s_w0),
        dict(idx_s=idx_s1, idx_d=idx_d1, buf_a=buf_a1, buf_b=buf_b1,
             buf_o=buf_o1,
             s_is=s_is1, s_id=s_id1, s_a=s_a1, s_b=s_b1, s_w=s_w1),
    )

    def start_l(k, sl):
        off = base + k * _CH
        pltpu.async_copy(src_hbm.at[pl.ds(off, _CH)], sl["idx_s"], sl["s_is"])
        pltpu.async_copy(dst_hbm.at[pl.ds(off, _CH)], sl["idx_d"], sl["s_id"])

    def wait_l(sl):
        pltpu.make_async_copy(src_hbm.at[pl.ds(base, _CH)], sl["idx_s"],
                              sl["s_is"]).wait()
        pltpu.make_async_copy(dst_hbm.at[pl.ds(base, _CH)], sl["idx_d"],
                              sl["s_id"]).wait()

    def start_g(sl):
        pltpu.async_copy(p_hbm.at[sl["idx_s"]], sl["buf_a"], sl["s_a"])
        pltpu.async_copy(q_hbm.at[sl["idx_d"]], sl["buf_b"], sl["s_b"])

    def wait_g(sl):
        pltpu.make_async_copy(p_hbm.at[sl["idx_s"]], sl["buf_a"],
                              sl["s_a"]).wait()
        pltpu.make_async_copy(q_hbm.at[sl["idx_d"]], sl["buf_b"],
                              sl["s_b"]).wait()

    def start_w(k, sl):
        pltpu.async_copy(sl["buf_o"], g_hbm.at[pl.ds(base + k * _CH, _CH)],
                         sl["s_w"])

    def wait_w(sl):
        pltpu.make_async_copy(sl["buf_o"], g_hbm.at[pl.ds(base, _CH)],
                              sl["s_w"]).wait()

    def vadd(sl):
        buf_a, buf_b, buf_o = sl["buf_a"], sl["buf_b"], sl["buf_o"]

        def row(r, rc):
            for cc in range(0, h, 16):
                buf_o[r, pl.ds(cc, 16)] = (buf_a[r, pl.ds(cc, 16)]
                                           + buf_b[r, pl.ds(cc, 16)])
            return rc

        lax.fori_loop(0, _CH, row, 0)

    start_l(0, slots[0])
    wait_l(slots[0])
    start_g(slots[0])
    start_l(1, slots[1])

    def step(k, sl, osl):
        @pl.when(k + 1 < nch)
        def _():
            wait_l(osl)

        @pl.when(k + 1 < nch)
        def _():
            start_g(osl)

        wait_g(sl)

        @pl.when(k + 2 < nch)
        def _():
            start_l(k + 2, sl)

        @pl.when(k >= 2)
        def _():
            wait_w(sl)

        vadd(sl)
        start_w(k, sl)

    def pair(i2, carry):
        k0 = 2 * i2

        @pl.when(k0 < nch)
        def _():
            step(k0, slots[0], slots[1])

        @pl.when(k0 + 1 < nch)
        def _():
            step(k0 + 1, slots[1], slots[0])

        return carry

    lax.fori_loop(0, (nch + 1) // 2, pair, 0)
    if nch >= 2:
        wait_w(slots[nch % 2])
    wait_w(slots[(nch - 1) % 2])


def _scatter_body(en_hbm, dst_hbm, aggp_hbm, agg_sh,
                  idx0, rows0, idx1, rows1,
                  s_i0, s_r0, s_s0, s_i1, s_r1, s_s1,
                  *, epw, nch, n, h):
    c = lax.axis_index("c")
    s = lax.axis_index("s")
    wid = s * _NC + c
    base = wid * epw
    slots = (
        dict(idx=idx0, rows=rows0, s_i=s_i0, s_r=s_r0, s_s=s_s0),
        dict(idx=idx1, rows=rows1, s_i=s_i1, s_r=s_r1, s_s=s_s1),
    )
    nzc = n // _CH
    zper = (nzc + _NS - 1) // _NS

    def zrow(r, rc):
        for cc in range(0, h, 16):
            rows0[r, pl.ds(cc, 16)] = jnp.zeros((16,), jnp.float32)
        return rc

    lax.fori_loop(0, _CH, zrow, 0)

    def zcp(j, rc):
        k = s + j * _NS

        @pl.when(k < nzc)
        def _():
            pltpu.sync_copy(rows0, agg_sh.at[pl.ds(k * _CH, _CH)])

        return rc

    lax.fori_loop(0, zper, zcp, 0)
    plsc.subcore_barrier()

    def start_l(k, sl):
        off = base + k * _CH
        pltpu.async_copy(dst_hbm.at[pl.ds(off, _CH)], sl["idx"], sl["s_i"])
        pltpu.async_copy(en_hbm.at[pl.ds(off, _CH)], sl["rows"], sl["s_r"])

    def wait_l(sl):
        pltpu.make_async_copy(dst_hbm.at[pl.ds(base, _CH)], sl["idx"],
                              sl["s_i"]).wait()
        pltpu.make_async_copy(en_hbm.at[pl.ds(base, _CH)], sl["rows"],
                              sl["s_r"]).wait()

    def start_s(sl):
        pltpu.async_copy(sl["rows"], agg_sh.at[sl["idx"]], sl["s_s"],
                         add=True)

    def wait_s(sl):
        pltpu.make_async_copy(sl["rows"], agg_sh.at[sl["idx"]],
                              sl["s_s"]).wait()

    start_l(0, slots[0])

    def step(k, sl, osl):
        wait_l(sl)
        start_s(sl)

        @pl.when(k >= 1)
        def _():
            wait_s(osl)

        @pl.when(k + 1 < nch)
        def _():
            start_l(k + 1, osl)

    def pair(i2, carry):
        k0 = 2 * i2

        @pl.when(k0 < nch)
        def _():
            step(k0, slots[0], slots[1])

        @pl.when(k0 + 1 < nch)
        def _():
            step(k0 + 1, slots[1], slots[0])

        return carry

    lax.fori_loop(0, (nch + 1) // 2, pair, 0)
    wait_s(slots[(nch - 1) % 2])
    plsc.subcore_barrier()

    def wcp(j, rc):
        k = s + j * _NS

        @pl.when(k < nzc)
        def _():
            pltpu.sync_copy(agg_sh.at[pl.ds(k * _CH, _CH)],
                            aggp_hbm.at[pl.ds(c * n + k * _CH, _CH)])

        return rc

    lax.fori_loop(0, zper, wcp, 0)


def kernel(x, edge_index, e, Wb1, bb1, Wb2, bb2, Wa1, ba1, Wa2, ba2):
    n, h = x.shape
    eN = e.shape[0]
    assert eN % _NW == 0
    epw = eN // _NW
    assert epw % _CH == 0
    nch = epw // _CH
    assert n % _NS == 0 and h % 16 == 0

    src = edge_index[0]
    dst = edge_index[1]
    Wb1a, Wb1b, Wb1c = Wb1[:h], Wb1[h:2 * h], Wb1[2 * h:]
    Wa1a, Wa1b = Wa1[:h], Wa1[h:]
    bb1r = bb1.reshape(1, h)
    bb2r = bb2.reshape(1, h)
    ba1r = ba1.reshape(1, h)
    ba2r = ba2.reshape(1, h)

    nb = 1000
    P, Q = pl.pallas_call(
        _pq_body,
        grid=(n // nb,),
        in_specs=[
            pl.BlockSpec((nb, h), lambda i: (i, 0)),
            pl.BlockSpec((h, h), lambda i: (0, 0)),
            pl.BlockSpec((h, h), lambda i: (0, 0)),
        ],
        out_specs=[
            pl.BlockSpec((nb, h), lambda i: (i, 0)),
            pl.BlockSpec((nb, h), lambda i: (i, 0)),
        ],
        out_shape=[
            jax.ShapeDtypeStruct((n, h), jnp.float32),
            jax.ShapeDtypeStruct((n, h), jnp.float32),
        ],
    )(x, Wb1a, Wb1b)

    mesh = plsc.VectorSubcoreMesh(core_axis_name="c", subcore_axis_name="s")
    gather_k = pl.kernel(
        functools.partial(_gather_body, epw=epw, nch=nch, h=h),
        out_type=jax.ShapeDtypeStruct((eN, h), jnp.float32),
        mesh=mesh,
        scratch_types=(
            [pltpu.VMEM((_CH,), jnp.int32), pltpu.VMEM((_CH,), jnp.int32),
             pltpu.VMEM((_CH, h), jnp.float32), pltpu.VMEM((_CH, h), jnp.float32),
             pltpu.VMEM((_CH, h), jnp.float32)] * 2
            + [pltpu.SemaphoreType.DMA] * 10
        ),
    )
    G = gather_k(P, Q, src, dst)

    eb = 3200
    e_new = pl.pallas_call(
        _edge_body,
        grid=(eN // eb,),
        in_specs=[
            pl.BlockSpec((eb, h), lambda i: (i, 0)),
            pl.BlockSpec((eb, h), lambda i: (i, 0)),
            pl.BlockSpec((h, h), lambda i: (0, 0)),
            pl.BlockSpec((1, h), lambda i: (0, 0)),
            pl.BlockSpec((h, h), lambda i: (0, 0)),
            pl.BlockSpec((1, h), lambda i: (0, 0)),
        ],
        out_specs=pl.BlockSpec((eb, h), lambda i: (i, 0)),
        out_shape=jax.ShapeDtypeStruct((eN, h), jnp.float32),
    )(e, G, Wb1c, bb1r, Wb2, bb2r)

    scatter_k = pl.kernel(
        functools.partial(_scatter_body, epw=epw, nch=nch, n=n, h=h),
        out_type=jax.ShapeDtypeStruct((_NC * n, h), jnp.float32),
        mesh=mesh,
        scratch_types=(
            [pltpu.VMEM_SHARED((n, h), jnp.float32)]
            + [pltpu.VMEM((_CH,), jnp.int32), pltpu.VMEM((_CH, h), jnp.float32)] * 2
            + [pltpu.SemaphoreType.DMA] * 6
        ),
    )
    aggp = scatter_k(e_new, dst)
    agg_parts = [aggp[:n], aggp[n:]]

    x_new = pl.pallas_call(
        _node_body,
        grid=(n // nb,),
        in_specs=[pl.BlockSpec((nb, h), lambda i: (i, 0))]
        + [pl.BlockSpec((nb, h), lambda i: (i, 0))] * len(agg_parts)
        + [
            pl.BlockSpec((h, h), lambda i: (0, 0)),
            pl.BlockSpec((h, h), lambda i: (0, 0)),
            pl.BlockSpec((1, h), lambda i: (0, 0)),
            pl.BlockSpec((h, h), lambda i: (0, 0)),
            pl.BlockSpec((1, h), lambda i: (0, 0)),
        ],
        out_specs=pl.BlockSpec((nb, h), lambda i: (i, 0)),
        out_shape=jax.ShapeDtypeStruct((n, h), jnp.float32),
    )(x, *agg_parts, Wa1a, Wa1b, ba1r, Wa2, ba2r)

    return (x_new, e_new)


# speedup vs baseline: 3.0685x; 1.0739x over previous
import functools

import jax
import jax.numpy as jnp
from jax import lax
from jax.experimental import pallas as pl
from jax.experimental.pallas import tpu as pltpu
from jax.experimental.pallas import tpu_sc as plsc

_NC = 2
_NS = 16
_NW = _NC * _NS
_CH = 80


def _pq_body(x_ref, wa_ref, wb_ref, p_ref, q_ref):
    xb = x_ref[...]
    p_ref[...] = jnp.dot(xb, wa_ref[...], preferred_element_type=jnp.float32)
    q_ref[...] = jnp.dot(xb, wb_ref[...], preferred_element_type=jnp.float32)


def _edge_body(e_ref, g_ref, w1_ref, b1_ref, w2_ref, b2_ref, out_ref):
    h = jnp.dot(e_ref[...], w1_ref[...], preferred_element_type=jnp.float32)
    h = jnp.maximum(h + g_ref[...] + b1_ref[...], 0.0)
    out_ref[...] = jnp.dot(h, w2_ref[...], preferred_element_type=jnp.float32) + b2_ref[...]


def _node_body(x_ref, *refs):
    w1a_ref, w1b_ref, b1_ref, w2_ref, b2_ref, out_ref = refs[-6:]
    agg_refs = refs[:-6]
    agg = agg_refs[0][...]
    for a_ref in agg_refs[1:]:
        agg = agg + a_ref[...]
    h = jnp.dot(x_ref[...], w1a_ref[...], preferred_element_type=jnp.float32)
    h = h + jnp.dot(agg, w1b_ref[...], preferred_element_type=jnp.float32)
    h = jnp.maximum(h + b1_ref[...], 0.0)
    out_ref[...] = jnp.dot(h, w2_ref[...], preferred_element_type=jnp.float32) + b2_ref[...]


def _gather_body(p_hbm, q_hbm, src_hbm, dst_hbm, g_hbm,
                 idx_s0, idx_d0, buf_a0, buf_b0, buf_o0,
                 idx_s1, idx_d1, buf_a1, buf_b1, buf_o1,
                 s_is0, s_id0, s_a0, s_b0, s_w0,
                 s_is1, s_id1, s_a1, s_b1, s_w1, *, epw, nch, h):
    c = lax.axis_index("c")
    s = lax.axis_index("s")
    base = (s * _NC + c) * epw
    slots = (
        dict(idx_s=idx_s0, idx_d=idx_d0, buf_a=buf_a0, buf_b=buf_b0,
             buf_o=buf_o0,
             s_is=s_is0, s_id=s_id0, s_a=s_a0, s_b=s_b0, s_w=s_w0),
        dict(idx_s=idx_s1, idx_d=idx_d1, buf_a=buf_a1, buf_b=buf_b1,
             buf_o=buf_o1,
             s_is=s_is1, s_id=s_id1, s_a=s_a1, s_b=s_b1, s_w=s_w1),
    )

    def start_l(k, sl):
        off = base + k * _CH
        pltpu.async_copy(src_hbm.at[pl.ds(off, _CH)], sl["idx_s"], sl["s_is"])
        pltpu.async_copy(dst_hbm.at[pl.ds(off, _CH)], sl["idx_d"], sl["s_id"])

    def wait_l(sl):
        pltpu.make_async_copy(src_hbm.at[pl.ds(base, _CH)], sl["idx_s"],
                              sl["s_is"]).wait()
        pltpu.make_async_copy(dst_hbm.at[pl.ds(base, _CH)], sl["idx_d"],
                              sl["s_id"]).wait()

    def start_g(sl):
        pltpu.async_copy(p_hbm.at[sl["idx_s"]], sl["buf_a"], sl["s_a"])
        pltpu.async_copy(q_hbm.at[sl["idx_d"]], sl["buf_b"], sl["s_b"])

    def wait_g(sl):
        pltpu.make_async_copy(p_hbm.at[sl["idx_s"]], sl["buf_a"],
                              sl["s_a"]).wait()
        pltpu.make_async_copy(q_hbm.at[sl["idx_d"]], sl["buf_b"],
                              sl["s_b"]).wait()

    def start_w(k, sl):
        pltpu.async_copy(sl["buf_o"], g_hbm.at[pl.ds(base + k * _CH, _CH)],
                         sl["s_w"])

    def wait_w(sl):
        pltpu.make_async_copy(sl["buf_o"], g_hbm.at[pl.ds(base, _CH)],
                              sl["s_w"]).wait()

    def vadd(sl):
        buf_a, buf_b, buf_o = sl["buf_a"], sl["buf_b"], sl["buf_o"]

        def row(r, rc):
            for cc in range(0, h, 16):
                buf_o[r, pl.ds(cc, 16)] = (buf_a[r, pl.ds(cc, 16)]
                                           + buf_b[r, pl.ds(cc, 16)])
            return rc

        lax.fori_loop(0, _CH, row, 0)

    start_l(0, slots[0])
    wait_l(slots[0])
    start_g(slots[0])
    start_l(1, slots[1])

    def step(k, sl, osl):
        @pl.when(k + 1 < nch)
        def _():
            wait_l(osl)

        @pl.when(k + 1 < nch)
        def _():
            start_g(osl)

        wait_g(sl)

        @pl.when(k + 2 < nch)
        def _():
            start_l(k + 2, sl)

        @pl.when(k >= 2)
        def _():
            wait_w(sl)

        vadd(sl)
        start_w(k, sl)

    def pair(i2, carry):
        k0 = 2 * i2

        @pl.when(k0 < nch)
        def _():
            step(k0, slots[0], slots[1])

        @pl.when(k0 + 1 < nch)
        def _():
            step(k0 + 1, slots[1], slots[0])

        return carry

    lax.fori_loop(0, (nch + 1) // 2, pair, 0)
    if nch >= 2:
        wait_w(slots[nch % 2])
    wait_w(slots[(nch - 1) % 2])


def _scatter_body(en_hbm, dst_hbm, aggp_hbm, agg_sh,
                  idx0, rows0, idx1, rows1, idx2, rows2, idx3, rows3,
                  s_i0, s_r0, s_s0, s_i1, s_r1, s_s1,
                  s_i2, s_r2, s_s2, s_i3, s_r3, s_s3,
                  *, epw, nch, n, h):
    c = lax.axis_index("c")
    s = lax.axis_index("s")
    wid = s * _NC + c
    base = wid * epw
    slots = (
        dict(idx=idx0, rows=rows0, s_i=s_i0, s_r=s_r0, s_s=s_s0),
        dict(idx=idx1, rows=rows1, s_i=s_i1, s_r=s_r1, s_s=s_s1),
        dict(idx=idx2, rows=rows2, s_i=s_i2, s_r=s_r2, s_s=s_s2),
        dict(idx=idx3, rows=rows3, s_i=s_i3, s_r=s_r3, s_s=s_s3),
    )
    nsl = len(slots)

    def start_l(k, sl):
        off = base + k * _CH
        pltpu.async_copy(dst_hbm.at[pl.ds(off, _CH)], sl["idx"], sl["s_i"])
        pltpu.async_copy(en_hbm.at[pl.ds(off, _CH)], sl["rows"], sl["s_r"])

    def wait_l(sl):
        pltpu.make_async_copy(dst_hbm.at[pl.ds(base, _CH)], sl["idx"],
                              sl["s_i"]).wait()
        pltpu.make_async_copy(en_hbm.at[pl.ds(base, _CH)], sl["rows"],
                              sl["s_r"]).wait()

    def start_s(sl):
        pltpu.async_copy(sl["rows"], agg_sh.at[sl["idx"]], sl["s_s"],
                         add=True)

    def wait_s(sl):
        pltpu.make_async_copy(sl["rows"], agg_sh.at[sl["idx"]],
                              sl["s_s"]).wait()

    start_l(0, slots[0])
    start_l(1, slots[1])
    zbuf = rows3

    nzc = n // _CH
    zper = (nzc + _NS - 1) // _NS

    def zrow(r, rc):
        for cc in range(0, h, 16):
            zbuf[r, pl.ds(cc, 16)] = jnp.zeros((16,), jnp.float32)
        return rc

    lax.fori_loop(0, _CH, zrow, 0)

    def zcp(j, rc):
        k = s + j * _NS

        @pl.when(k < nzc)
        def _():
            pltpu.sync_copy(zbuf, agg_sh.at[pl.ds(k * _CH, _CH)])

        return rc

    lax.fori_loop(0, zper, zcp, 0)
    plsc.subcore_barrier()

    def step(k, sl, fsl):
        wait_l(sl)
        start_s(sl)

        @pl.when(k >= 2)
        def _():
            wait_s(fsl)

        @pl.when(k + 2 < nch)
        def _():
            start_l(k + 2, fsl)

    def quad(i4, carry):
        k0 = 4 * i4
        for j in range(4):
            k = k0 + j

            @pl.when(k < nch)
            def _(k=k, j=j):
                step(k, slots[j], slots[(j + 2) % nsl])

        return carry

    lax.fori_loop(0, (nch + nsl - 1) // nsl, quad, 0)
    wait_s(slots[(nch - 2) % nsl])
    wait_s(slots[(nch - 1) % nsl])
    plsc.subcore_barrier()

    def wcp(j, rc):
        k = s + j * _NS

        @pl.when(k < nzc)
        def _():
            pltpu.sync_copy(agg_sh.at[pl.ds(k * _CH, _CH)],
                            aggp_hbm.at[pl.ds(c * n + k * _CH, _CH)])

        return rc

    lax.fori_loop(0, zper, wcp, 0)


def kernel(x, edge_index, e, Wb1, bb1, Wb2, bb2, Wa1, ba1, Wa2, ba2):
    n, h = x.shape
    eN = e.shape[0]
    assert eN % _NW == 0
    epw = eN // _NW
    assert epw % _CH == 0
    nch = epw // _CH
    assert n % _NS == 0 and h % 16 == 0

    src = edge_index[0]
    dst = edge_index[1]
    Wb1a, Wb1b, Wb1c = Wb1[:h], Wb1[h:2 * h], Wb1[2 * h:]
    Wa1a, Wa1b = Wa1[:h], Wa1[h:]
    bb1r = bb1.reshape(1, h)
    bb2r = bb2.reshape(1, h)
    ba1r = ba1.reshape(1, h)
    ba2r = ba2.reshape(1, h)

    nb = 1000
    P, Q = pl.pallas_call(
        _pq_body,
        grid=(n // nb,),
        in_specs=[
            pl.BlockSpec((nb, h), lambda i: (i, 0)),
            pl.BlockSpec((h, h), lambda i: (0, 0)),
            pl.BlockSpec((h, h), lambda i: (0, 0)),
        ],
        out_specs=[
            pl.BlockSpec((nb, h), lambda i: (i, 0)),
            pl.BlockSpec((nb, h), lambda i: (i, 0)),
        ],
        out_shape=[
            jax.ShapeDtypeStruct((n, h), jnp.float32),
            jax.ShapeDtypeStruct((n, h), jnp.float32),
        ],
    )(x, Wb1a, Wb1b)

    mesh = plsc.VectorSubcoreMesh(core_axis_name="c", subcore_axis_name="s")
    gather_k = pl.kernel(
        functools.partial(_gather_body, epw=epw, nch=nch, h=h),
        out_type=jax.ShapeDtypeStruct((eN, h), jnp.float32),
        mesh=mesh,
        scratch_types=(
            [pltpu.VMEM((_CH,), jnp.int32), pltpu.VMEM((_CH,), jnp.int32),
             pltpu.VMEM((_CH, h), jnp.float32), pltpu.VMEM((_CH, h), jnp.float32),
             pltpu.VMEM((_CH, h), jnp.float32)] * 2
            + [pltpu.SemaphoreType.DMA] * 10
        ),
    )
    G = gather_k(P, Q, src, dst)

    eb = 3200
    e_new = pl.pallas_call(
        _edge_body,
        grid=(eN // eb,),
        in_specs=[
            pl.BlockSpec((eb, h), lambda i: (i, 0)),
            pl.BlockSpec((eb, h), lambda i: (i, 0)),
            pl.BlockSpec((h, h), lambda i: (0, 0)),
            pl.BlockSpec((1, h), lambda i: (0, 0)),
            pl.BlockSpec((h, h), lambda i: (0, 0)),
            pl.BlockSpec((1, h), lambda i: (0, 0)),
        ],
        out_specs=pl.BlockSpec((eb, h), lambda i: (i, 0)),
        out_shape=jax.ShapeDtypeStruct((eN, h), jnp.float32),
    )(e, G, Wb1c, bb1r, Wb2, bb2r)

    scatter_k = pl.kernel(
        functools.partial(_scatter_body, epw=epw, nch=nch, n=n, h=h),
        out_type=jax.ShapeDtypeStruct((_NC * n, h), jnp.float32),
        mesh=mesh,
        scratch_types=(
            [pltpu.VMEM_SHARED((n, h), jnp.float32)]
            + [pltpu.VMEM((_CH,), jnp.int32), pltpu.VMEM((_CH, h), jnp.float32)] * 4
            + [pltpu.SemaphoreType.DMA] * 12
        ),
    )
    aggp = scatter_k(e_new, dst)
    agg_parts = [aggp[:n], aggp[n:]]

    x_new = pl.pallas_call(
        _node_body,
        grid=(n // nb,),
        in_specs=[pl.BlockSpec((nb, h), lambda i: (i, 0))]
        + [pl.BlockSpec((nb, h), lambda i: (i, 0))] * len(agg_parts)
        + [
            pl.BlockSpec((h, h), lambda i: (0, 0)),
            pl.BlockSpec((h, h), lambda i: (0, 0)),
            pl.BlockSpec((1, h), lambda i: (0, 0)),
            pl.BlockSpec((h, h), lambda i: (0, 0)),
            pl.BlockSpec((1, h), lambda i: (0, 0)),
        ],
        out_specs=pl.BlockSpec((nb, h), lambda i: (i, 0)),
        out_shape=jax.ShapeDtypeStruct((n, h), jnp.float32),
    )(x, *agg_parts, Wa1a, Wa1b, ba1r, Wa2, ba2r)

    return (x_new, e_new)


# speedup vs baseline: 3.0735x; 1.0016x over previous
import functools

import jax
import jax.numpy as jnp
from jax import lax
from jax.experimental import pallas as pl
from jax.experimental.pallas import tpu as pltpu
from jax.experimental.pallas import tpu_sc as plsc

_NC = 2
_NS = 16
_NW = _NC * _NS
_CH = 80


def _pq_body(x_ref, wa_ref, wb_ref, p_ref, q_ref):
    xb = x_ref[...]
    p_ref[...] = jnp.dot(xb, wa_ref[...], preferred_element_type=jnp.float32)
    q_ref[...] = jnp.dot(xb, wb_ref[...], preferred_element_type=jnp.float32)


def _edge_body(e_ref, g_ref, w1_ref, b1_ref, w2_ref, b2_ref, out_ref):
    h = jnp.dot(e_ref[...], w1_ref[...], preferred_element_type=jnp.float32)
    h = jnp.maximum(h + g_ref[...] + b1_ref[...], 0.0)
    out_ref[...] = jnp.dot(h, w2_ref[...], preferred_element_type=jnp.float32) + b2_ref[...]


def _node_body(x_ref, *refs):
    w1a_ref, w1b_ref, b1_ref, w2_ref, b2_ref, out_ref = refs[-6:]
    agg_refs = refs[:-6]
    agg = agg_refs[0][...]
    for a_ref in agg_refs[1:]:
        agg = agg + a_ref[...]
    h = jnp.dot(x_ref[...], w1a_ref[...], preferred_element_type=jnp.float32)
    h = h + jnp.dot(agg, w1b_ref[...], preferred_element_type=jnp.float32)
    h = jnp.maximum(h + b1_ref[...], 0.0)
    out_ref[...] = jnp.dot(h, w2_ref[...], preferred_element_type=jnp.float32) + b2_ref[...]


def _gather_body(p_hbm, q_hbm, src_hbm, dst_hbm, g_hbm,
                 idx_s0, idx_d0, buf_a0, buf_b0, buf_o0,
                 idx_s1, idx_d1, buf_a1, buf_b1, buf_o1,
                 idx_s2, idx_d2, buf_a2, buf_b2, buf_o2,
                 s_is0, s_id0, s_a0, s_b0, s_w0,
                 s_is1, s_id1, s_a1, s_b1, s_w1,
                 s_is2, s_id2, s_a2, s_b2, s_w2, *, epw, nch, h):
    c = lax.axis_index("c")
    s = lax.axis_index("s")
    base = (s * _NC + c) * epw
    slots = (
        dict(idx_s=idx_s0, idx_d=idx_d0, buf_a=buf_a0, buf_b=buf_b0,
             buf_o=buf_o0,
             s_is=s_is0, s_id=s_id0, s_a=s_a0, s_b=s_b0, s_w=s_w0),
        dict(idx_s=idx_s1, idx_d=idx_d1, buf_a=buf_a1, buf_b=buf_b1,
             buf_o=buf_o1,
             s_is=s_is1, s_id=s_id1, s_a=s_a1, s_b=s_b1, s_w=s_w1),
        dict(idx_s=idx_s2, idx_d=idx_d2, buf_a=buf_a2, buf_b=buf_b2,
             buf_o=buf_o2,
             s_is=s_is2, s_id=s_id2, s_a=s_a2, s_b=s_b2, s_w=s_w2),
    )
    nsl = len(slots)

    def start_l(k, sl):
        off = base + k * _CH
        pltpu.async_copy(src_hbm.at[pl.ds(off, _CH)], sl["idx_s"], sl["s_is"])
        pltpu.async_copy(dst_hbm.at[pl.ds(off, _CH)], sl["idx_d"], sl["s_id"])

    def wait_l(sl):
        pltpu.make_async_copy(src_hbm.at[pl.ds(base, _CH)], sl["idx_s"],
                              sl["s_is"]).wait()
        pltpu.make_async_copy(dst_hbm.at[pl.ds(base, _CH)], sl["idx_d"],
                              sl["s_id"]).wait()

    def start_g(sl):
        pltpu.async_copy(p_hbm.at[sl["idx_s"]], sl["buf_a"], sl["s_a"])
        pltpu.async_copy(q_hbm.at[sl["idx_d"]], sl["buf_b"], sl["s_b"])

    def wait_g(sl):
        pltpu.make_async_copy(p_hbm.at[sl["idx_s"]], sl["buf_a"],
                              sl["s_a"]).wait()
        pltpu.make_async_copy(q_hbm.at[sl["idx_d"]], sl["buf_b"],
                              sl["s_b"]).wait()

    def start_w(k, sl):
        pltpu.async_copy(sl["buf_o"], g_hbm.at[pl.ds(base + k * _CH, _CH)],
                         sl["s_w"])

    def wait_w(sl):
        pltpu.make_async_copy(sl["buf_o"], g_hbm.at[pl.ds(base, _CH)],
                              sl["s_w"]).wait()

    def vadd(sl):
        buf_a, buf_b, buf_o = sl["buf_a"], sl["buf_b"], sl["buf_o"]

        def row(j, rc):
            r = 2 * j
            for rr in range(2):
                for cc in range(0, h, 16):
                    buf_o[r + rr, pl.ds(cc, 16)] = (
                        buf_a[r + rr, pl.ds(cc, 16)]
                        + buf_b[r + rr, pl.ds(cc, 16)])
            return rc

        lax.fori_loop(0, _CH // 2, row, 0)

    start_l(0, slots[0])
    start_l(1, slots[1])
    start_l(2, slots[2])
    wait_l(slots[0])
    start_g(slots[0])

    def step(k, sl, nsl_):
        @pl.when(k + 1 < nch)
        def _():
            wait_l(nsl_)
            start_g(nsl_)

        wait_g(sl)

        @pl.when(k + 3 < nch)
        def _():
            start_l(k + 3, sl)

        @pl.when(k >= 3)
        def _():
            wait_w(sl)

        vadd(sl)
        start_w(k, sl)

    def triple(i3, carry):
        k0 = 3 * i3
        for j in range(3):
            k = k0 + j

            @pl.when(k < nch)
            def _(k=k, j=j):
                step(k, slots[j], slots[(j + 1) % nsl])

        return carry

    lax.fori_loop(0, (nch + nsl - 1) // nsl, triple, 0)
    for t in range(min(3, nch)):
        wait_w(slots[(nch - 1 - t) % nsl])


def _scatter_body(en_hbm, dst_hbm, aggp_hbm, agg_sh,
                  idx0, rows0, idx1, rows1, idx2, rows2, idx3, rows3,
                  s_i0, s_r0, s_s0, s_i1, s_r1, s_s1,
                  s_i2, s_r2, s_s2, s_i3, s_r3, s_s3,
                  *, epw, nch, n, h):
    c = lax.axis_index("c")
    s = lax.axis_index("s")
    wid = s * _NC + c
    base = wid * epw
    slots = (
        dict(idx=idx0, rows=rows0, s_i=s_i0, s_r=s_r0, s_s=s_s0),
        dict(idx=idx1, rows=rows1, s_i=s_i1, s_r=s_r1, s_s=s_s1),
        dict(idx=idx2, rows=rows2, s_i=s_i2, s_r=s_r2, s_s=s_s2),
        dict(idx=idx3, rows=rows3, s_i=s_i3, s_r=s_r3, s_s=s_s3),
    )
    nsl = len(slots)

    def start_l(k, sl):
        off = base + k * _CH
        pltpu.async_copy(dst_hbm.at[pl.ds(off, _CH)], sl["idx"], sl["s_i"])
        pltpu.async_copy(en_hbm.at[pl.ds(off, _CH)], sl["rows"], sl["s_r"])

    def wait_l(sl):
        pltpu.make_async_copy(dst_hbm.at[pl.ds(base, _CH)], sl["idx"],
                              sl["s_i"]).wait()
        pltpu.make_async_copy(en_hbm.at[pl.ds(base, _CH)], sl["rows"],
                              sl["s_r"]).wait()

    def start_s(sl):
        pltpu.async_copy(sl["rows"], agg_sh.at[sl["idx"]], sl["s_s"],
                         add=True)

    def wait_s(sl):
        pltpu.make_async_copy(sl["rows"], agg_sh.at[sl["idx"]],
                              sl["s_s"]).wait()

    start_l(0, slots[0])
    start_l(1, slots[1])
    zbuf = rows3

    nzc = n // _CH
    zper = (nzc + _NS - 1) // _NS

    def zrow(r, rc):
        for cc in range(0, h, 16):
            zbuf[r, pl.ds(cc, 16)] = jnp.zeros((16,), jnp.float32)
        return rc

    lax.fori_loop(0, _CH, zrow, 0)

    def zcp(j, rc):
        k = s + j * _NS

        @pl.when(k < nzc)
        def _():
            pltpu.sync_copy(zbuf, agg_sh.at[pl.ds(k * _CH, _CH)])

        return rc

    lax.fori_loop(0, zper, zcp, 0)
    plsc.subcore_barrier()

    def step(k, sl, fsl):
        wait_l(sl)
        start_s(sl)

        @pl.when(k >= 2)
        def _():
            wait_s(fsl)

        @pl.when(k + 2 < nch)
        def _():
            start_l(k + 2, fsl)

    def quad(i4, carry):
        k0 = 4 * i4
        for j in range(4):
            k = k0 + j

            @pl.when(k < nch)
            def _(k=k, j=j):
                step(k, slots[j], slots[(j + 2) % nsl])

        return carry

    lax.fori_loop(0, (nch + nsl - 1) // nsl, quad, 0)
    wait_s(slots[(nch - 2) % nsl])
    wait_s(slots[(nch - 1) % nsl])
    plsc.subcore_barrier()

    def wcp(j, rc):
        k = s + j * _NS

        @pl.when(k < nzc)
        def _():
            pltpu.sync_copy(agg_sh.at[pl.ds(k * _CH, _CH)],
                            aggp_hbm.at[pl.ds(c * n + k * _CH, _CH)])

        return rc

    lax.fori_loop(0, zper, wcp, 0)


def kernel(x, edge_index, e, Wb1, bb1, Wb2, bb2, Wa1, ba1, Wa2, ba2):
    n, h = x.shape
    eN = e.shape[0]
    assert eN % _NW == 0
    epw = eN // _NW
    assert epw % _CH == 0
    nch = epw // _CH
    assert n % _NS == 0 and h % 16 == 0

    src = edge_index[0]
    dst = edge_index[1]
    Wb1a, Wb1b, Wb1c = Wb1[:h], Wb1[h:2 * h], Wb1[2 * h:]
    Wa1a, Wa1b = Wa1[:h], Wa1[h:]
    bb1r = bb1.reshape(1, h)
    bb2r = bb2.reshape(1, h)
    ba1r = ba1.reshape(1, h)
    ba2r = ba2.reshape(1, h)

    nb = 1000
    P, Q = pl.pallas_call(
        _pq_body,
        grid=(n // nb,),
        in_specs=[
            pl.BlockSpec((nb, h), lambda i: (i, 0)),
            pl.BlockSpec((h, h), lambda i: (0, 0)),
            pl.BlockSpec((h, h), lambda i: (0, 0)),
        ],
        out_specs=[
            pl.BlockSpec((nb, h), lambda i: (i, 0)),
            pl.BlockSpec((nb, h), lambda i: (i, 0)),
        ],
        out_shape=[
            jax.ShapeDtypeStruct((n, h), jnp.float32),
            jax.ShapeDtypeStruct((n, h), jnp.float32),
        ],
    )(x, Wb1a, Wb1b)

    mesh = plsc.VectorSubcoreMesh(core_axis_name="c", subcore_axis_name="s")
    gather_k = pl.kernel(
        functools.partial(_gather_body, epw=epw, nch=nch, h=h),
        out_type=jax.ShapeDtypeStruct((eN, h), jnp.float32),
        mesh=mesh,
        scratch_types=(
            [pltpu.VMEM((_CH,), jnp.int32), pltpu.VMEM((_CH,), jnp.int32),
             pltpu.VMEM((_CH, h), jnp.float32), pltpu.VMEM((_CH, h), jnp.float32),
             pltpu.VMEM((_CH, h), jnp.float32)] * 3
            + [pltpu.SemaphoreType.DMA] * 15
        ),
    )
    G = gather_k(P, Q, src, dst)

    eb = 3200
    e_new = pl.pallas_call(
        _edge_body,
        grid=(eN // eb,),
        in_specs=[
            pl.BlockSpec((eb, h), lambda i: (i, 0)),
            pl.BlockSpec((eb, h), lambda i: (i, 0)),
            pl.BlockSpec((h, h), lambda i: (0, 0)),
            pl.BlockSpec((1, h), lambda i: (0, 0)),
            pl.BlockSpec((h, h), lambda i: (0, 0)),
            pl.BlockSpec((1, h), lambda i: (0, 0)),
        ],
        out_specs=pl.BlockSpec((eb, h), lambda i: (i, 0)),
        out_shape=jax.ShapeDtypeStruct((eN, h), jnp.float32),
    )(e, G, Wb1c, bb1r, Wb2, bb2r)

    scatter_k = pl.kernel(
        functools.partial(_scatter_body, epw=epw, nch=nch, n=n, h=h),
        out_type=jax.ShapeDtypeStruct((_NC * n, h), jnp.float32),
        mesh=mesh,
        scratch_types=(
            [pltpu.VMEM_SHARED((n, h), jnp.float32)]
            + [pltpu.VMEM((_CH,), jnp.int32), pltpu.VMEM((_CH, h), jnp.float32)] * 4
            + [pltpu.SemaphoreType.DMA] * 12
        ),
    )
    aggp = scatter_k(e_new, dst)
    agg_parts = [aggp[:n], aggp[n:]]

    x_new = pl.pallas_call(
        _node_body,
        grid=(n // nb,),
        in_specs=[pl.BlockSpec((nb, h), lambda i: (i, 0))]
        + [pl.BlockSpec((nb, h), lambda i: (i, 0))] * len(agg_parts)
        + [
            pl.BlockSpec((h, h), lambda i: (0, 0)),
            pl.BlockSpec((h, h), lambda i: (0, 0)),
            pl.BlockSpec((1, h), lambda i: (0, 0)),
            pl.BlockSpec((h, h), lambda i: (0, 0)),
            pl.BlockSpec((1, h), lambda i: (0, 0)),
        ],
        out_specs=pl.BlockSpec((nb, h), lambda i: (i, 0)),
        out_shape=jax.ShapeDtypeStruct((n, h), jnp.float32),
    )(x, *agg_parts, Wa1a, Wa1b, ba1r, Wa2, ba2r)

    return (x_new, e_new)


# speedup vs baseline: 3.2388x; 1.0538x over previous
import functools

import jax
import jax.numpy as jnp
from jax import lax
from jax.experimental import pallas as pl
from jax.experimental.pallas import tpu as pltpu
from jax.experimental.pallas import tpu_sc as plsc

_NC = 2
_NS = 16
_NW = _NC * _NS
_CH = 80


def _pq_body(x_ref, wa_ref, wb_ref, p_ref, q_ref):
    xb = x_ref[...]
    p_ref[...] = jnp.dot(xb, wa_ref[...], preferred_element_type=jnp.float32)
    q_ref[...] = jnp.dot(xb, wb_ref[...], preferred_element_type=jnp.float32)


def _edge_body(e_ref, g_ref, w1_ref, b1_ref, w2_ref, b2_ref, out_ref):
    h = jnp.dot(e_ref[...], w1_ref[...], preferred_element_type=jnp.float32)
    h = jnp.maximum(h + g_ref[...] + b1_ref[...], 0.0)
    out_ref[...] = jnp.dot(h, w2_ref[...], preferred_element_type=jnp.float32) + b2_ref[...]


def _node_body(x_ref, *refs):
    w1a_ref, w1b_ref, b1_ref, w2_ref, b2_ref, out_ref = refs[-6:]
    agg_refs = refs[:-6]
    agg = agg_refs[0][...]
    for a_ref in agg_refs[1:]:
        agg = agg + a_ref[...]
    h = jnp.dot(x_ref[...], w1a_ref[...], preferred_element_type=jnp.float32)
    h = h + jnp.dot(agg, w1b_ref[...], preferred_element_type=jnp.float32)
    h = jnp.maximum(h + b1_ref[...], 0.0)
    out_ref[...] = jnp.dot(h, w2_ref[...], preferred_element_type=jnp.float32) + b2_ref[...]


def _gather_body(p_hbm, q_hbm, src_hbm, dst_hbm, g_hbm,
                 idx_s0, idx_d0, buf_a0, buf_b0, buf_o0,
                 idx_s1, idx_d1, buf_a1, buf_b1, buf_o1,
                 idx_s2, idx_d2, buf_a2, buf_b2, buf_o2,
                 s_is0, s_id0, s_a0, s_b0, s_w0,
                 s_is1, s_id1, s_a1, s_b1, s_w1,
                 s_is2, s_id2, s_a2, s_b2, s_w2, *, epw, nch, h):
    c = lax.axis_index("c")
    s = lax.axis_index("s")
    base = (s * _NC + c) * epw
    slots = (
        dict(idx_s=idx_s0, idx_d=idx_d0, buf_a=buf_a0, buf_b=buf_b0,
             buf_o=buf_o0,
             s_is=s_is0, s_id=s_id0, s_a=s_a0, s_b=s_b0, s_w=s_w0),
        dict(idx_s=idx_s1, idx_d=idx_d1, buf_a=buf_a1, buf_b=buf_b1,
             buf_o=buf_o1,
             s_is=s_is1, s_id=s_id1, s_a=s_a1, s_b=s_b1, s_w=s_w1),
        dict(idx_s=idx_s2, idx_d=idx_d2, buf_a=buf_a2, buf_b=buf_b2,
             buf_o=buf_o2,
             s_is=s_is2, s_id=s_id2, s_a=s_a2, s_b=s_b2, s_w=s_w2),
    )
    nsl = len(slots)

    def start_l(k, sl):
        off = base + k * _CH
        pltpu.async_copy(src_hbm.at[pl.ds(off, _CH)], sl["idx_s"], sl["s_is"])
        pltpu.async_copy(dst_hbm.at[pl.ds(off, _CH)], sl["idx_d"], sl["s_id"])

    def wait_l(sl):
        pltpu.make_async_copy(src_hbm.at[pl.ds(base, _CH)], sl["idx_s"],
                              sl["s_is"]).wait()
        pltpu.make_async_copy(dst_hbm.at[pl.ds(base, _CH)], sl["idx_d"],
                              sl["s_id"]).wait()

    def start_g(sl):
        pltpu.async_copy(p_hbm.at[sl["idx_s"]], sl["buf_a"], sl["s_a"])
        pltpu.async_copy(q_hbm.at[sl["idx_d"]], sl["buf_b"], sl["s_b"])

    def wait_g(sl):
        pltpu.make_async_copy(p_hbm.at[sl["idx_s"]], sl["buf_a"],
                              sl["s_a"]).wait()
        pltpu.make_async_copy(q_hbm.at[sl["idx_d"]], sl["buf_b"],
                              sl["s_b"]).wait()

    def start_w(k, sl):
        pltpu.async_copy(sl["buf_o"], g_hbm.at[pl.ds(base + k * _CH, _CH)],
                         sl["s_w"])

    def wait_w(sl):
        pltpu.make_async_copy(sl["buf_o"], g_hbm.at[pl.ds(base, _CH)],
                              sl["s_w"]).wait()

    def vadd(sl):
        buf_a, buf_b, buf_o = sl["buf_a"], sl["buf_b"], sl["buf_o"]

        def row(j, rc):
            r = 2 * j
            for rr in range(2):
                for cc in range(0, h, 16):
                    buf_o[r + rr, pl.ds(cc, 16)] = (
                        buf_a[r + rr, pl.ds(cc, 16)]
                        + buf_b[r + rr, pl.ds(cc, 16)])
            return rc

        lax.fori_loop(0, _CH // 2, row, 0)

    start_l(0, slots[0])
    start_l(1, slots[1])
    start_l(2, slots[2])
    wait_l(slots[0])
    start_g(slots[0])

    def step(k, sl, nsl_):
        @pl.when(k + 1 < nch)
        def _():
            wait_l(nsl_)
            start_g(nsl_)

        wait_g(sl)

        @pl.when(k + 3 < nch)
        def _():
            start_l(k + 3, sl)

        @pl.when(k >= 3)
        def _():
            wait_w(sl)

        vadd(sl)
        start_w(k, sl)

    def triple(i3, carry):
        k0 = 3 * i3
        for j in range(3):
            k = k0 + j

            @pl.when(k < nch)
            def _(k=k, j=j):
                step(k, slots[j], slots[(j + 1) % nsl])

        return carry

    lax.fori_loop(0, (nch + nsl - 1) // nsl, triple, 0)
    for t in range(min(3, nch)):
        wait_w(slots[(nch - 1 - t) % nsl])


def _scatter_body(en_hbm, dst_hbm, aggp_hbm, agg_sh,
                  idx0, rows0, idx1, rows1, idx2, rows2, idx3, rows3,
                  s_i0, s_r0, s_s0, s_i1, s_r1, s_s1,
                  s_i2, s_r2, s_s2, s_i3, s_r3, s_s3,
                  *, epw, nch, n, h):
    c = lax.axis_index("c")
    s = lax.axis_index("s")
    wid = s * _NC + c
    base = wid * epw
    slots = (
        dict(idx=idx0, rows=rows0, s_i=s_i0, s_r=s_r0, s_s=s_s0),
        dict(idx=idx1, rows=rows1, s_i=s_i1, s_r=s_r1, s_s=s_s1),
        dict(idx=idx2, rows=rows2, s_i=s_i2, s_r=s_r2, s_s=s_s2),
        dict(idx=idx3, rows=rows3, s_i=s_i3, s_r=s_r3, s_s=s_s3),
    )
    nsl = len(slots)

    def start_l(k, sl):
        off = base + k * _CH
        pltpu.async_copy(dst_hbm.at[pl.ds(off, _CH)], sl["idx"], sl["s_i"])
        pltpu.async_copy(en_hbm.at[pl.ds(off, _CH)], sl["rows"], sl["s_r"])

    def wait_l(sl):
        pltpu.make_async_copy(dst_hbm.at[pl.ds(base, _CH)], sl["idx"],
                              sl["s_i"]).wait()
        pltpu.make_async_copy(en_hbm.at[pl.ds(base, _CH)], sl["rows"],
                              sl["s_r"]).wait()

    def start_s(sl):
        pltpu.async_copy(sl["rows"], agg_sh.at[sl["idx"]], sl["s_s"],
                         add=True)

    def wait_s(sl):
        pltpu.make_async_copy(sl["rows"], agg_sh.at[sl["idx"]],
                              sl["s_s"]).wait()

    start_l(0, slots[0])
    start_l(1, slots[1])
    zbuf = rows3

    nzc = n // _CH
    zper = (nzc + _NS - 1) // _NS

    def zrow(r, rc):
        for cc in range(0, h, 16):
            zbuf[r, pl.ds(cc, 16)] = jnp.zeros((16,), jnp.float32)
        return rc

    lax.fori_loop(0, _CH, zrow, 0)

    def zcp(j, rc):
        k = s + j * _NS

        @pl.when(k < nzc)
        def _():
            pltpu.sync_copy(zbuf, agg_sh.at[pl.ds(k * _CH, _CH)])

        return rc

    lax.fori_loop(0, zper, zcp, 0)
    plsc.subcore_barrier()

    def step(k, sl, fsl):
        wait_l(sl)
        start_s(sl)

        @pl.when(k >= 2)
        def _():
            wait_s(fsl)

        @pl.when(k + 2 < nch)
        def _():
            start_l(k + 2, fsl)

    def quad(i4, carry):
        k0 = 4 * i4
        for j in range(4):
            k = k0 + j

            @pl.when(k < nch)
            def _(k=k, j=j):
                step(k, slots[j], slots[(j + 2) % nsl])

        return carry

    lax.fori_loop(0, (nch + nsl - 1) // nsl, quad, 0)
    wait_s(slots[(nch - 2) % nsl])
    wait_s(slots[(nch - 1) % nsl])
    plsc.subcore_barrier()

    def wcp(j, rc):
        k = s + j * _NS

        @pl.when(k < nzc)
        def _():
            pltpu.sync_copy(agg_sh.at[pl.ds(k * _CH, _CH)],
                            aggp_hbm.at[pl.ds(c * n + k * _CH, _CH)])

        return rc

    lax.fori_loop(0, zper, wcp, 0)


def kernel(x, edge_index, e, Wb1, bb1, Wb2, bb2, Wa1, ba1, Wa2, ba2):
    n, h = x.shape
    eN = e.shape[0]
    assert eN % _NW == 0
    epw = eN // _NW
    assert epw % _CH == 0
    nch = epw // _CH
    assert n % _NS == 0 and h % 16 == 0

    src = edge_index[0]
    dst = edge_index[1]
    Wb1a, Wb1b, Wb1c = Wb1[:h], Wb1[h:2 * h], Wb1[2 * h:]
    Wa1a, Wa1b = Wa1[:h], Wa1[h:]
    bb1r = bb1.reshape(1, h)
    bb2r = bb2.reshape(1, h)
    ba1r = ba1.reshape(1, h)
    ba2r = ba2.reshape(1, h)

    nb = 2000
    P, Q = pl.pallas_call(
        _pq_body,
        grid=(n // nb,),
        in_specs=[
            pl.BlockSpec((nb, h), lambda i: (i, 0)),
            pl.BlockSpec((h, h), lambda i: (0, 0)),
            pl.BlockSpec((h, h), lambda i: (0, 0)),
        ],
        out_specs=[
            pl.BlockSpec((nb, h), lambda i: (i, 0)),
            pl.BlockSpec((nb, h), lambda i: (i, 0)),
        ],
        out_shape=[
            jax.ShapeDtypeStruct((n, h), jnp.float32),
            jax.ShapeDtypeStruct((n, h), jnp.float32),
        ],
    )(x, Wb1a, Wb1b)

    mesh = plsc.VectorSubcoreMesh(core_axis_name="c", subcore_axis_name="s")
    gather_k = pl.kernel(
        functools.partial(_gather_body, epw=epw, nch=nch, h=h),
        out_type=jax.ShapeDtypeStruct((eN, h), jnp.float32),
        mesh=mesh,
        scratch_types=(
            [pltpu.VMEM((_CH,), jnp.int32), pltpu.VMEM((_CH,), jnp.int32),
             pltpu.VMEM((_CH, h), jnp.float32), pltpu.VMEM((_CH, h), jnp.float32),
             pltpu.VMEM((_CH, h), jnp.float32)] * 3
            + [pltpu.SemaphoreType.DMA] * 15
        ),
    )
    G = gather_k(P, Q, src, dst)

    eb = 6400
    e_new = pl.pallas_call(
        _edge_body,
        grid=(eN // eb,),
        in_specs=[
            pl.BlockSpec((eb, h), lambda i: (i, 0)),
            pl.BlockSpec((eb, h), lambda i: (i, 0)),
            pl.BlockSpec((h, h), lambda i: (0, 0)),
            pl.BlockSpec((1, h), lambda i: (0, 0)),
            pl.BlockSpec((h, h), lambda i: (0, 0)),
            pl.BlockSpec((1, h), lambda i: (0, 0)),
        ],
        out_specs=pl.BlockSpec((eb, h), lambda i: (i, 0)),
        out_shape=jax.ShapeDtypeStruct((eN, h), jnp.float32),
    )(e, G, Wb1c, bb1r, Wb2, bb2r)

    scatter_k = pl.kernel(
        functools.partial(_scatter_body, epw=epw, nch=nch, n=n, h=h),
        out_type=jax.ShapeDtypeStruct((_NC * n, h), jnp.float32),
        mesh=mesh,
        scratch_types=(
            [pltpu.VMEM_SHARED((n, h), jnp.float32)]
            + [pltpu.VMEM((_CH,), jnp.int32), pltpu.VMEM((_CH, h), jnp.float32)] * 4
            + [pltpu.SemaphoreType.DMA] * 12
        ),
    )
    aggp = scatter_k(e_new, dst)
    agg_parts = [aggp[:n], aggp[n:]]

    x_new = pl.pallas_call(
        _node_body,
        grid=(n // nb,),
        in_specs=[pl.BlockSpec((nb, h), lambda i: (i, 0))]
        + [pl.BlockSpec((nb, h), lambda i: (i, 0))] * len(agg_parts)
        + [
            pl.BlockSpec((h, h), lambda i: (0, 0)),
            pl.BlockSpec((h, h), lambda i: (0, 0)),
            pl.BlockSpec((1, h), lambda i: (0, 0)),
            pl.BlockSpec((h, h), lambda i: (0, 0)),
            pl.BlockSpec((1, h), lambda i: (0, 0)),
        ],
        out_specs=pl.BlockSpec((nb, h), lambda i: (i, 0)),
        out_shape=jax.ShapeDtypeStruct((n, h), jnp.float32),
    )(x, *agg_parts, Wa1a, Wa1b, ba1r, Wa2, ba2r)

    return (x_new, e_new)


# speedup vs baseline: 3.2777x; 1.0120x over previous
import functools

import jax
import jax.numpy as jnp
from jax import lax
from jax.experimental import pallas as pl
from jax.experimental.pallas import tpu as pltpu
from jax.experimental.pallas import tpu_sc as plsc

_NC = 2
_NS = 16
_NW = _NC * _NS
_CH = 80


def _pq_body(x_ref, wa_ref, wb_ref, p_ref, q_ref):
    xb = x_ref[...]
    p_ref[...] = jnp.dot(xb, wa_ref[...], preferred_element_type=jnp.float32)
    q_ref[...] = jnp.dot(xb, wb_ref[...], preferred_element_type=jnp.float32)


def _edge_body(e_ref, g_ref, w1_ref, b1_ref, w2_ref, b2_ref, out_ref):
    h = jnp.dot(e_ref[...], w1_ref[...], preferred_element_type=jnp.float32)
    h = jnp.maximum(h + g_ref[...] + b1_ref[...], 0.0)
    out_ref[...] = jnp.dot(h, w2_ref[...], preferred_element_type=jnp.float32) + b2_ref[...]


def _node_body(x_ref, *refs):
    w1a_ref, w1b_ref, b1_ref, w2_ref, b2_ref, out_ref = refs[-6:]
    agg_refs = refs[:-6]
    agg = agg_refs[0][...]
    for a_ref in agg_refs[1:]:
        agg = agg + a_ref[...]
    h = jnp.dot(x_ref[...], w1a_ref[...], preferred_element_type=jnp.float32)
    h = h + jnp.dot(agg, w1b_ref[...], preferred_element_type=jnp.float32)
    h = jnp.maximum(h + b1_ref[...], 0.0)
    out_ref[...] = jnp.dot(h, w2_ref[...], preferred_element_type=jnp.float32) + b2_ref[...]


def _gather_body(p_hbm, q_hbm, src_hbm, dst_hbm, g_hbm,
                 idx_s0, idx_d0, buf_a0, buf_b0, buf_o0,
                 idx_s1, idx_d1, buf_a1, buf_b1, buf_o1,
                 idx_s2, idx_d2, buf_a2, buf_b2, buf_o2,
                 s_is0, s_id0, s_a0, s_b0, s_w0,
                 s_is1, s_id1, s_a1, s_b1, s_w1,
                 s_is2, s_id2, s_a2, s_b2, s_w2, *, epw, nch, h):
    c = lax.axis_index("c")
    s = lax.axis_index("s")
    base = (s * _NC + c) * epw
    slots = (
        dict(idx_s=idx_s0, idx_d=idx_d0, buf_a=buf_a0, buf_b=buf_b0,
             buf_o=buf_o0,
             s_is=s_is0, s_id=s_id0, s_a=s_a0, s_b=s_b0, s_w=s_w0),
        dict(idx_s=idx_s1, idx_d=idx_d1, buf_a=buf_a1, buf_b=buf_b1,
             buf_o=buf_o1,
             s_is=s_is1, s_id=s_id1, s_a=s_a1, s_b=s_b1, s_w=s_w1),
        dict(idx_s=idx_s2, idx_d=idx_d2, buf_a=buf_a2, buf_b=buf_b2,
             buf_o=buf_o2,
             s_is=s_is2, s_id=s_id2, s_a=s_a2, s_b=s_b2, s_w=s_w2),
    )
    nsl = len(slots)

    def start_l(k, sl):
        off = base + k * _CH
        pltpu.async_copy(src_hbm.at[pl.ds(off, _CH)], sl["idx_s"], sl["s_is"])
        pltpu.async_copy(dst_hbm.at[pl.ds(off, _CH)], sl["idx_d"], sl["s_id"])

    def wait_l(sl):
        pltpu.make_async_copy(src_hbm.at[pl.ds(base, _CH)], sl["idx_s"],
                              sl["s_is"]).wait()
        pltpu.make_async_copy(dst_hbm.at[pl.ds(base, _CH)], sl["idx_d"],
                              sl["s_id"]).wait()

    def start_g(sl):
        pltpu.async_copy(p_hbm.at[sl["idx_s"]], sl["buf_a"], sl["s_a"])
        pltpu.async_copy(q_hbm.at[sl["idx_d"]], sl["buf_b"], sl["s_b"])

    def wait_g(sl):
        pltpu.make_async_copy(p_hbm.at[sl["idx_s"]], sl["buf_a"],
                              sl["s_a"]).wait()
        pltpu.make_async_copy(q_hbm.at[sl["idx_d"]], sl["buf_b"],
                              sl["s_b"]).wait()

    def start_w(k, sl):
        pltpu.async_copy(sl["buf_o"], g_hbm.at[pl.ds(base + k * _CH, _CH)],
                         sl["s_w"])

    def wait_w(sl):
        pltpu.make_async_copy(sl["buf_o"], g_hbm.at[pl.ds(base, _CH)],
                              sl["s_w"]).wait()

    def vadd(sl):
        buf_a, buf_b, buf_o = sl["buf_a"], sl["buf_b"], sl["buf_o"]

        def row(j, rc):
            r = 2 * j
            for rr in range(2):
                for cc in range(0, h, 16):
                    buf_o[r + rr, pl.ds(cc, 16)] = (
                        buf_a[r + rr, pl.ds(cc, 16)]
                        + buf_b[r + rr, pl.ds(cc, 16)])
            return rc

        lax.fori_loop(0, _CH // 2, row, 0)

    start_l(0, slots[0])
    start_l(1, slots[1])
    start_l(2, slots[2])
    wait_l(slots[0])
    start_g(slots[0])

    def step(k, sl, nsl_):
        @pl.when(k + 1 < nch)
        def _():
            wait_l(nsl_)
            start_g(nsl_)

        wait_g(sl)

        @pl.when(k + 3 < nch)
        def _():
            start_l(k + 3, sl)

        @pl.when(k >= 3)
        def _():
            wait_w(sl)

        vadd(sl)
        start_w(k, sl)

    def triple(i3, carry):
        k0 = 3 * i3
        for j in range(3):
            k = k0 + j

            @pl.when(k < nch)
            def _(k=k, j=j):
                step(k, slots[j], slots[(j + 1) % nsl])

        return carry

    lax.fori_loop(0, (nch + nsl - 1) // nsl, triple, 0)
    for t in range(min(3, nch)):
        wait_w(slots[(nch - 1 - t) % nsl])


def _scatter_body(en_hbm, dst_hbm, aggp_hbm, agg_sh,
                  idx0, rows0, idx1, rows1, idx2, rows2, idx3, rows3,
                  s_i0, s_r0, s_s0, s_i1, s_r1, s_s1,
                  s_i2, s_r2, s_s2, s_i3, s_r3, s_s3,
                  *, epw, nch, n, h):
    c = lax.axis_index("c")
    s = lax.axis_index("s")
    wid = s * _NC + c
    base = wid * epw
    slots = (
        dict(idx=idx0, rows=rows0, s_i=s_i0, s_r=s_r0, s_s=s_s0),
        dict(idx=idx1, rows=rows1, s_i=s_i1, s_r=s_r1, s_s=s_s1),
        dict(idx=idx2, rows=rows2, s_i=s_i2, s_r=s_r2, s_s=s_s2),
        dict(idx=idx3, rows=rows3, s_i=s_i3, s_r=s_r3, s_s=s_s3),
    )
    nsl = len(slots)

    def start_l(k, sl):
        off = base + k * _CH
        pltpu.async_copy(dst_hbm.at[pl.ds(off, _CH)], sl["idx"], sl["s_i"])
        pltpu.async_copy(en_hbm.at[pl.ds(off, _CH)], sl["rows"], sl["s_r"])

    def wait_l(sl):
        pltpu.make_async_copy(dst_hbm.at[pl.ds(base, _CH)], sl["idx"],
                              sl["s_i"]).wait()
        pltpu.make_async_copy(en_hbm.at[pl.ds(base, _CH)], sl["rows"],
                              sl["s_r"]).wait()

    def start_s(sl):
        pltpu.async_copy(sl["rows"], agg_sh.at[sl["idx"]], sl["s_s"],
                         add=True)

    def wait_s(sl):
        pltpu.make_async_copy(sl["rows"], agg_sh.at[sl["idx"]],
                              sl["s_s"]).wait()

    start_l(0, slots[0])
    start_l(1, slots[1])
    zbuf = rows3

    nzc = n // _CH
    zper = (nzc + _NS - 1) // _NS

    def zrow(r, rc):
        for cc in range(0, h, 16):
            zbuf[r, pl.ds(cc, 16)] = jnp.zeros((16,), jnp.float32)
        return rc

    lax.fori_loop(0, _CH, zrow, 0)

    def zcp(j, rc):
        k = s + j * _NS

        @pl.when(k < nzc)
        def _():
            pltpu.sync_copy(zbuf, agg_sh.at[pl.ds(k * _CH, _CH)])

        return rc

    lax.fori_loop(0, zper, zcp, 0)
    plsc.subcore_barrier()

    def step(k, sl, fsl):
        wait_l(sl)
        start_s(sl)

        @pl.when(k >= 2)
        def _():
            wait_s(fsl)

        @pl.when(k + 2 < nch)
        def _():
            start_l(k + 2, fsl)

    def quad(i4, carry):
        k0 = 4 * i4
        for j in range(4):
            k = k0 + j

            @pl.when(k < nch)
            def _(k=k, j=j):
                step(k, slots[j], slots[(j + 2) % nsl])

        return carry

    lax.fori_loop(0, (nch + nsl - 1) // nsl, quad, 0)
    wait_s(slots[(nch - 2) % nsl])
    wait_s(slots[(nch - 1) % nsl])
    plsc.subcore_barrier()

    def wcp(j, rc):
        k = s + j * _NS

        @pl.when(k < nzc)
        def _():
            pltpu.sync_copy(agg_sh.at[pl.ds(k * _CH, _CH)],
                            aggp_hbm.at[pl.ds(c * n + k * _CH, _CH)])

        return rc

    lax.fori_loop(0, zper, wcp, 0)


def kernel(x, edge_index, e, Wb1, bb1, Wb2, bb2, Wa1, ba1, Wa2, ba2):
    n, h = x.shape
    eN = e.shape[0]
    assert eN % _NW == 0
    epw = eN // _NW
    assert epw % _CH == 0
    nch = epw // _CH
    assert n % _NS == 0 and h % 16 == 0

    src = edge_index[0]
    dst = edge_index[1]
    Wb1a, Wb1b, Wb1c = Wb1[:h], Wb1[h:2 * h], Wb1[2 * h:]
    Wa1a, Wa1b = Wa1[:h], Wa1[h:]
    bb1r = bb1.reshape(1, h)
    bb2r = bb2.reshape(1, h)
    ba1r = ba1.reshape(1, h)
    ba2r = ba2.reshape(1, h)

    nb = 2000
    P, Q = pl.pallas_call(
        _pq_body,
        grid=(n // nb,),
        in_specs=[
            pl.BlockSpec((nb, h), lambda i: (i, 0)),
            pl.BlockSpec((h, h), lambda i: (0, 0)),
            pl.BlockSpec((h, h), lambda i: (0, 0)),
        ],
        out_specs=[
            pl.BlockSpec((nb, h), lambda i: (i, 0)),
            pl.BlockSpec((nb, h), lambda i: (i, 0)),
        ],
        out_shape=[
            jax.ShapeDtypeStruct((n, h), jnp.float32),
            jax.ShapeDtypeStruct((n, h), jnp.float32),
        ],
    )(x, Wb1a, Wb1b)

    mesh = plsc.VectorSubcoreMesh(core_axis_name="c", subcore_axis_name="s")
    gather_k = pl.kernel(
        functools.partial(_gather_body, epw=epw, nch=nch, h=h),
        out_type=jax.ShapeDtypeStruct((eN, h), jnp.float32),
        mesh=mesh,
        scratch_types=(
            [pltpu.VMEM((_CH,), jnp.int32), pltpu.VMEM((_CH,), jnp.int32),
             pltpu.VMEM((_CH, h), jnp.float32), pltpu.VMEM((_CH, h), jnp.float32),
             pltpu.VMEM((_CH, h), jnp.float32)] * 3
            + [pltpu.SemaphoreType.DMA] * 15
        ),
    )
    G = gather_k(P, Q, src, dst)

    eb = 12800
    e_new = pl.pallas_call(
        _edge_body,
        grid=(eN // eb,),
        in_specs=[
            pl.BlockSpec((eb, h), lambda i: (i, 0)),
            pl.BlockSpec((eb, h), lambda i: (i, 0)),
            pl.BlockSpec((h, h), lambda i: (0, 0)),
            pl.BlockSpec((1, h), lambda i: (0, 0)),
            pl.BlockSpec((h, h), lambda i: (0, 0)),
            pl.BlockSpec((1, h), lambda i: (0, 0)),
        ],
        out_specs=pl.BlockSpec((eb, h), lambda i: (i, 0)),
        out_shape=jax.ShapeDtypeStruct((eN, h), jnp.float32),
    )(e, G, Wb1c, bb1r, Wb2, bb2r)

    scatter_k = pl.kernel(
        functools.partial(_scatter_body, epw=epw, nch=nch, n=n, h=h),
        out_type=jax.ShapeDtypeStruct((_NC * n, h), jnp.float32),
        mesh=mesh,
        scratch_types=(
            [pltpu.VMEM_SHARED((n, h), jnp.float32)]
            + [pltpu.VMEM((_CH,), jnp.int32), pltpu.VMEM((_CH, h), jnp.float32)] * 4
            + [pltpu.SemaphoreType.DMA] * 12
        ),
    )
    aggp = scatter_k(e_new, dst)
    agg_parts = [aggp[:n], aggp[n:]]

    x_new = pl.pallas_call(
        _node_body,
        grid=(n // nb,),
        in_specs=[pl.BlockSpec((nb, h), lambda i: (i, 0))]
        + [pl.BlockSpec((nb, h), lambda i: (i, 0))] * len(agg_parts)
        + [
            pl.BlockSpec((h, h), lambda i: (0, 0)),
            pl.BlockSpec((h, h), lambda i: (0, 0)),
            pl.BlockSpec((1, h), lambda i: (0, 0)),
            pl.BlockSpec((h, h), lambda i: (0, 0)),
            pl.BlockSpec((1, h), lambda i: (0, 0)),
        ],
        out_specs=pl.BlockSpec((nb, h), lambda i: (i, 0)),
        out_shape=jax.ShapeDtypeStruct((n, h), jnp.float32),
    )(x, *agg_parts, Wa1a, Wa1b, ba1r, Wa2, ba2r)

    return (x_new, e_new)
